# baseline (speedup 1.0000x reference)
import functools

import jax
import jax.numpy as jnp
from jax import lax
from jax.experimental import pallas as pl
from jax.experimental.pallas import tpu as pltpu
from jax.experimental.pallas import tpu_sc as plsc

N = 10000
H = 32
K = 15
CIN = 128
COUT = 128
KP_EXTENT = 1.2
E = N * H

NC = 2
NS = 16
NW = NC * NS

CH = 128
NCHUNK = E // CH
MAXC = (NCHUNK + NW - 1) // NW

B = 200
BH = B * H
GB = N // B
G = 8
NG = B // G


def _sc_gather_body(table_hbm, inds_hbm, xn_hbm, idx_v, rows_v, sem):
    wid = lax.axis_index("s") * NC + lax.axis_index("c")

    def body(i, carry):
        c = wid + i * NW

        @pl.when(c < NCHUNK)
        def _():
            off = pl.multiple_of(c * CH, CH)
            pltpu.sync_copy(inds_hbm.at[pl.ds(off, CH)], idx_v)
            pltpu.async_copy(table_hbm.at[idx_v], rows_v, sem).wait()
            pltpu.sync_copy(rows_v, xn_hbm.at[pl.ds(off, CH)])

        return carry

    lax.fori_loop(0, MAXC, body, 0)


def _sc_gather(table, inds):
    mesh = plsc.VectorSubcoreMesh(core_axis_name="c", subcore_axis_name="s")
    fn = pl.kernel(
        _sc_gather_body,
        mesh=mesh,
        out_type=jax.ShapeDtypeStruct((E, CIN), jnp.float32),
        scratch_types=[
            pltpu.VMEM((CH,), jnp.int32),
            pltpu.VMEM((CH, CIN), jnp.float32),
            pltpu.SemaphoreType.DMA,
        ],
    )
    return fn(table, inds)


def _tc_body(q_ref, kaug_ref, w_ref, xn_ref, out_ref):
    raw = xn_ref[...]
    wi = lax.bitcast_convert_type(raw[:, 0:64], jnp.int32)
    f_even = lax.bitcast_convert_type(
        wi & jnp.int32(-65536), jnp.float32)
    f_odd = lax.bitcast_convert_type(wi << 16, jnp.float32)
    feats = jnp.concatenate([f_even, f_odd], axis=1)
    c3 = raw[:, 64:67]
    q = q_ref[...]
    qb = jnp.broadcast_to(q[:, None, :], (B, H, 3)).reshape(BH, 3)
    n3 = c3 - qb
    n3sq = jnp.sum(n3 * n3, axis=1, keepdims=True)
    ones = jnp.ones((BH, 1), jnp.float32)
    n3aug = jnp.concatenate([n3, n3sq, ones], axis=1)
    sqt = lax.dot_general(kaug_ref[...], n3aug, (((1,), (1,)), ((), ())),
                          preferred_element_type=jnp.float32)
    wgtt = jnp.maximum(
        1.0 - jnp.sqrt(jnp.maximum(sqt, 0.0)) * (1.0 / KP_EXTENT), 0.0)
    r_i = lax.broadcasted_iota(jnp.int32, (G * K, G * H), 0)
    c_i = lax.broadcasted_iota(jnp.int32, (G * K, G * H), 1)
    maskf = ((r_i // K) == (c_i // H)).astype(jnp.float32)
    parts = []
    for g in range(NG):
        wg = wgtt[:, g * G * H:(g + 1) * G * H]
        rep = jnp.tile(wg, (G, 1))
        bdiag = rep * maskf
        fg = feats[g * G * H:(g + 1) * G * H, :]
        parts.append(jnp.dot(bdiag, fg, preferred_element_type=jnp.float32))
    a2 = jnp.concatenate(parts, axis=0)
    a = a2.reshape(B, K * CIN)
    out_ref[...] = jnp.dot(a, w_ref[...], preferred_element_type=jnp.float32)


def _tc_call(q_pts, kaug, wperm, xn):
    return pl.pallas_call(
        _tc_body,
        grid=(GB,),
        in_specs=[
            pl.BlockSpec((B, 3), lambda i: (i, 0)),
            pl.BlockSpec((K, 5), lambda i: (0, 0)),
            pl.BlockSpec((K * CIN, COUT), lambda i: (0, 0)),
            pl.BlockSpec((BH, CIN), lambda i: (i, 0)),
        ],
        out_specs=pl.BlockSpec((B, COUT), lambda i: (i, 0)),
        out_shape=jax.ShapeDtypeStruct((N, COUT), jnp.float32),
    )(q_pts, kaug, wperm, xn)


def _pack_table(x, s_pts):
    xb = x.astype(jnp.bfloat16)
    u = lax.bitcast_convert_type(xb, jnp.uint16).astype(jnp.uint32)
    w = (u[:, 0::2] << 16) | u[:, 1::2]
    packedf = lax.bitcast_convert_type(w, jnp.float32)
    row = jnp.concatenate(
        [packedf, s_pts, jnp.zeros((N, 61), jnp.float32)], axis=1)
    shadow = jnp.zeros((1, 128), jnp.float32).at[0, 64:67].set(1e6)
    return jnp.concatenate([row, shadow], axis=0)


def kernel(q_pts, s_pts, neighb_inds, x, K_points, W):
    table = _pack_table(x, s_pts)
    inds = neighb_inds.astype(jnp.int32).reshape(E)
    xn = _sc_gather(table, inds)
    kaug = jnp.concatenate(
        [-2.0 * K_points,
         jnp.ones((K, 1), jnp.float32),
         jnp.sum(K_points * K_points, axis=1, keepdims=True)], axis=1)
    perm = jnp.arange(CIN).reshape(64, 2).T.reshape(CIN)
    wperm = W[:, perm, :].reshape(K * CIN, COUT)
    return _tc_call(q_pts, kaug, wperm, xn)


# speedup vs baseline: 1.0146x; 1.0146x over previous
import functools

import jax
import jax.numpy as jnp
from jax import lax
from jax.experimental import pallas as pl
from jax.experimental.pallas import tpu as pltpu
from jax.experimental.pallas import tpu_sc as plsc

N = 10000
H = 32
K = 15
KP = 16
CIN = 128
COUT = 128
KP_EXTENT = 1.2
E = N * H

NC = 2
NS = 16
NW = NC * NS

CH = 128
NCHUNK = E // CH
MAXC = (NCHUNK + NW - 1) // NW

B = 200
BH = B * H
GB = N // B


def _sc_gather_body(table_hbm, inds_hbm, xn_hbm, idx_v, rows_v, sem):
    wid = lax.axis_index("s") * NC + lax.axis_index("c")

    def body(i, carry):
        c = wid + i * NW

        @pl.when(c < NCHUNK)
        def _():
            off = pl.multiple_of(c * CH, CH)
            pltpu.sync_copy(inds_hbm.at[pl.ds(off, CH)], idx_v)
            pltpu.async_copy(table_hbm.at[idx_v], rows_v, sem).wait()
            pltpu.sync_copy(rows_v, xn_hbm.at[pl.ds(off, CH)])

        return carry

    lax.fori_loop(0, MAXC, body, 0)


def _sc_gather(table, inds):
    mesh = plsc.VectorSubcoreMesh(core_axis_name="c", subcore_axis_name="s")
    fn = pl.kernel(
        _sc_gather_body,
        mesh=mesh,
        out_type=jax.ShapeDtypeStruct((E, CIN), jnp.float32),
        scratch_types=[
            pltpu.VMEM((CH,), jnp.int32),
            pltpu.VMEM((CH, CIN), jnp.float32),
            pltpu.SemaphoreType.DMA,
        ],
    )
    return fn(table, inds)


def _tc_body(t_ref, w_ref, xn_ref, out_ref):
    raw = xn_ref[...]
    wi = lax.bitcast_convert_type(raw[:, 0:64], jnp.int32)
    f_even = lax.bitcast_convert_type(wi << 16, jnp.float32)
    f_odd = lax.bitcast_convert_type(wi & jnp.int32(-65536), jnp.float32)
    feats = jnp.concatenate([f_even, f_odd], axis=1)
    tb = t_ref[...]
    sx = raw[:, 64:65].reshape(B, H, 1)
    sy = raw[:, 65:66].reshape(B, H, 1)
    sz = raw[:, 66:67].reshape(B, H, 1)
    s2 = raw[:, 67:68].reshape(B, H, 1)
    txe = tb[:, None, 0:16]
    tye = tb[:, None, 16:32]
    tze = tb[:, None, 32:48]
    twe = tb[:, None, 48:64]
    sq3 = s2 + twe + sx * txe + sy * tye + sz * tze
    wgt3 = jnp.maximum(
        1.0 - jnp.sqrt(jnp.maximum(sq3, 0.0)) * (1.0 / KP_EXTENT), 0.0)
    f3 = feats.reshape(B, H, CIN)
    a3 = lax.dot_general(wgt3, f3, (((1,), (1,)), ((0,), (0,))),
                         preferred_element_type=jnp.float32)
    a = a3.reshape(B, KP * CIN)
    out_ref[...] = jnp.dot(a, w_ref[...], preferred_element_type=jnp.float32)


def _tc_call(taug, wflat, xn):
    return pl.pallas_call(
        _tc_body,
        grid=(GB,),
        in_specs=[
            pl.BlockSpec((B, 64), lambda i: (i, 0)),
            pl.BlockSpec((KP * CIN, COUT), lambda i: (0, 0)),
            pl.BlockSpec((BH, CIN), lambda i: (i, 0)),
        ],
        out_specs=pl.BlockSpec((B, COUT), lambda i: (i, 0)),
        out_shape=jax.ShapeDtypeStruct((N, COUT), jnp.float32),
    )(taug, wflat, xn)


def _pack_table(x, s_pts):
    xb = x.astype(jnp.bfloat16)
    packedf = lax.bitcast_convert_type(
        xb.reshape(N, 64, 2), jnp.float32)
    s2 = jnp.sum(s_pts * s_pts, axis=1, keepdims=True)
    row = jnp.concatenate(
        [packedf, s_pts, s2, jnp.zeros((N, 60), jnp.float32)], axis=1)
    shadow = jnp.zeros((1, 128), jnp.float32)
    shadow = shadow.at[0, 64:67].set(1e6).at[0, 67].set(3e12)
    return jnp.concatenate([row, shadow], axis=0)


def _make_taug(q_pts, K_points):
    t = q_pts[:, None, :] + K_points[None, :, :]
    pad0 = jnp.zeros((N, 1), jnp.float32)
    padw = jnp.full((N, 1), 1e30, jnp.float32)
    tx = jnp.concatenate([-2.0 * t[..., 0], pad0], axis=1)
    ty = jnp.concatenate([-2.0 * t[..., 1], pad0], axis=1)
    tz = jnp.concatenate([-2.0 * t[..., 2], pad0], axis=1)
    tw = jnp.concatenate([jnp.sum(t * t, axis=2), padw], axis=1)
    return jnp.concatenate([tx, ty, tz, tw], axis=1)


def kernel(q_pts, s_pts, neighb_inds, x, K_points, W):
    table = _pack_table(x, s_pts)
    inds = neighb_inds.astype(jnp.int32).reshape(E)
    xn = _sc_gather(table, inds)
    taug = _make_taug(q_pts, K_points)
    perm = jnp.arange(CIN).reshape(64, 2).T.reshape(CIN)
    wperm = W[:, perm, :]
    wflat = jnp.concatenate(
        [wperm, jnp.zeros((1, CIN, COUT), jnp.float32)],
        axis=0).reshape(KP * CIN, COUT)
    return _tc_call(taug, wflat, xn)
